# baseline (speedup 1.0000x reference)
import functools

import jax
import jax.numpy as jnp
from jax.experimental import pallas as pl
from jax.experimental.pallas import tpu as pltpu

_THRESHOLD = 0.5
_NEG_POS_RATIO = 3.0
_ALPHA = 1.0
_BISECT_ITERS = 30


def _mbl_kernel(plocs_ref, scores_ref, boxes_ref, labels_ref, priors_ref,
                out_ref, acc_loc, acc_np, acc_cp, acc_hn,
                *, n_b, n_obj, n_cls, n_p):
    i = pl.program_id(0)

    bv = boxes_ref[...].reshape(n_obj, 4)
    lab = labels_ref[...].reshape(n_obj, 1)
    pt = priors_ref[...]
    cx = pt[0:1, :]
    cy = pt[1:2, :]
    w = pt[2:3, :]
    h = pt[3:4, :]
    px1 = cx - w * 0.5
    py1 = cy - h * 0.5
    px2 = cx + w * 0.5
    py2 = cy + h * 0.5

    bx1 = bv[:, 0:1]
    by1 = bv[:, 1:2]
    bx2 = bv[:, 2:3]
    by2 = bv[:, 3:4]

    iw = jnp.maximum(jnp.minimum(bx2, px2) - jnp.maximum(bx1, px1), 0.0)
    ih = jnp.maximum(jnp.minimum(by2, py2) - jnp.maximum(by1, py1), 0.0)
    inter = iw * ih
    area_b = (bx2 - bx1) * (by2 - by1)
    area_p = (px2 - px1) * (py2 - py1)
    ov = inter / (area_b + area_p - inter)

    obj_iota = jax.lax.broadcasted_iota(jnp.int32, (n_obj, n_p), 0)
    col_iota = jax.lax.broadcasted_iota(jnp.int32, (n_obj, n_p), 1)

    ovmax_p = jnp.max(ov, axis=0, keepdims=True)
    ofep = jnp.min(jnp.where(ov == ovmax_p, obj_iota, n_obj),
                   axis=0, keepdims=True)
    rowmax = jnp.max(ov, axis=1, keepdims=True)
    pfeo = jnp.min(jnp.where(ov == rowmax, col_iota, n_p),
                   axis=1, keepdims=True)

    is_best = col_iota == pfeo
    j_assign = jnp.max(jnp.where(is_best, obj_iota, -1),
                       axis=0, keepdims=True)
    forced = j_assign >= 0
    ofep = jnp.where(forced, j_assign, ofep)
    ovfep = jnp.where(forced, 1.0, ovmax_p)

    sel = obj_iota == ofep
    lfp = jnp.sum(jnp.where(sel, lab, 0), axis=0, keepdims=True)
    tc = jnp.where(ovfep < _THRESHOLD, 0, lfp)
    pos = tc != 0
    posf = pos.astype(jnp.float32)
    npos_i = jnp.sum(posf, axis=1, keepdims=True)

    zf = jnp.zeros((), jnp.float32)
    g_x1 = jnp.sum(jnp.where(sel, bx1, zf), axis=0, keepdims=True)
    g_y1 = jnp.sum(jnp.where(sel, by1, zf), axis=0, keepdims=True)
    g_x2 = jnp.sum(jnp.where(sel, bx2, zf), axis=0, keepdims=True)
    g_y2 = jnp.sum(jnp.where(sel, by2, zf), axis=0, keepdims=True)

    gcx = (g_x1 + g_x2) * 0.5
    gcy = (g_y1 + g_y2) * 0.5
    gw = g_x2 - g_x1
    gh = g_y2 - g_y1
    t0 = (gcx - cx) / (w / 10.0)
    t1 = (gcy - cy) / (h / 10.0)
    t2 = jnp.log(gw / w) * 5.0
    t3 = jnp.log(gh / h) * 5.0

    plc = plocs_ref[...].reshape(4, n_p)
    ad = (jnp.abs(plc[0:1, :] - t0) + jnp.abs(plc[1:2, :] - t1)
          + jnp.abs(plc[2:3, :] - t2) + jnp.abs(plc[3:4, :] - t3))
    loc_i = jnp.sum(jnp.where(pos, ad, zf), axis=1, keepdims=True)

    st = scores_ref[...].reshape(n_p, n_cls)
    tc_col = jnp.transpose(tc)
    m = jnp.max(st, axis=1, keepdims=True)
    lse = jnp.log(jnp.sum(jnp.exp(st - m), axis=1, keepdims=True)) + m
    cls_iota = jax.lax.broadcasted_iota(jnp.int32, (n_p, n_cls), 1)
    sal = jnp.sum(jnp.where(cls_iota == tc_col, st, zf),
                  axis=1, keepdims=True)
    ce = jnp.transpose(lse - sal)

    cp_i = jnp.sum(jnp.where(pos, ce, zf), axis=1, keepdims=True)
    v = jnp.where(pos, zf, ce)

    kf = _NEG_POS_RATIO * npos_i
    lo = jnp.zeros((1, 1), jnp.float32)
    hi = jnp.max(v, axis=1, keepdims=True)
    for _ in range(_BISECT_ITERS):
        mid = (lo + hi) * 0.5
        c = jnp.sum((v > mid).astype(jnp.float32), axis=1, keepdims=True)
        take_lo = c >= kf
        lo = jnp.where(take_lo, mid, lo)
        hi = jnp.where(take_lo, hi, mid)
    above = v > hi
    cnt = jnp.sum(above.astype(jnp.float32), axis=1, keepdims=True)
    hn_i = (jnp.sum(jnp.where(above, v, zf), axis=1, keepdims=True)
            + (kf - cnt) * hi)

    @pl.when(i == 0)
    def _():
        acc_loc[...] = loc_i
        acc_np[...] = npos_i
        acc_cp[...] = cp_i
        acc_hn[...] = hn_i

    @pl.when(i > 0)
    def _():
        acc_loc[...] = acc_loc[...] + loc_i
        acc_np[...] = acc_np[...] + npos_i
        acc_cp[...] = acc_cp[...] + cp_i
        acc_hn[...] = acc_hn[...] + hn_i

    @pl.when(i == n_b - 1)
    def _():
        npos_t = acc_np[...]
        conf = (acc_hn[...] + acc_cp[...]) / npos_t
        loc = acc_loc[...] / (npos_t * 4.0)
        out_ref[...] = conf + _ALPHA * loc


def kernel(predicted_locs, predicted_scores, boxes, labels, priors_cxcy):
    n_b, n_p, n_cls = predicted_scores.shape
    n_obj = boxes.shape[1]
    plocs_t = jnp.transpose(predicted_locs, (0, 2, 1))
    priors_t = jnp.transpose(priors_cxcy, (1, 0))
    labels3 = labels.astype(jnp.int32)[..., None]

    out = pl.pallas_call(
        functools.partial(_mbl_kernel, n_b=n_b, n_obj=n_obj,
                          n_cls=n_cls, n_p=n_p),
        grid=(n_b,),
        in_specs=[
            pl.BlockSpec((1, 4, n_p), lambda i: (i, 0, 0)),
            pl.BlockSpec((1, n_p, n_cls), lambda i: (i, 0, 0)),
            pl.BlockSpec((1, n_obj, 4), lambda i: (i, 0, 0)),
            pl.BlockSpec((1, n_obj, 1), lambda i: (i, 0, 0)),
            pl.BlockSpec((4, n_p), lambda i: (0, 0)),
        ],
        out_specs=pl.BlockSpec((1, 1), lambda i: (0, 0)),
        out_shape=jax.ShapeDtypeStruct((1, 1), jnp.float32),
        scratch_shapes=[pltpu.VMEM((1, 1), jnp.float32)] * 4,
    )(plocs_t, predicted_scores, boxes, labels3, priors_t)
    return out[0, 0]


# speedup vs baseline: 1.3947x; 1.3947x over previous
import functools

import jax
import jax.numpy as jnp
from jax.experimental import pallas as pl
from jax.experimental.pallas import tpu as pltpu

_THRESHOLD = 0.5
_NEG_POS_RATIO = 3.0
_ALPHA = 1.0
_BISECT_ITERS = 30


def _mbl_kernel(plocs_ref, scores_ref, boxes_ref, labels_ref, priors_ref,
                out_ref, acc_loc, acc_np, acc_cp, acc_hn,
                *, n_b, n_obj, n_cls, n_p):
    i = pl.program_id(0)

    bv = boxes_ref[...].reshape(n_obj, 4)
    lab = labels_ref[...].reshape(n_obj, 1)
    pt = priors_ref[...]
    cx = pt[0:1, :]
    cy = pt[1:2, :]
    w = pt[2:3, :]
    h = pt[3:4, :]
    px1 = cx - w * 0.5
    py1 = cy - h * 0.5
    px2 = cx + w * 0.5
    py2 = cy + h * 0.5

    bx1 = bv[:, 0:1]
    by1 = bv[:, 1:2]
    bx2 = bv[:, 2:3]
    by2 = bv[:, 3:4]

    iw = jnp.maximum(jnp.minimum(bx2, px2) - jnp.maximum(bx1, px1), 0.0)
    ih = jnp.maximum(jnp.minimum(by2, py2) - jnp.maximum(by1, py1), 0.0)
    inter = iw * ih
    area_b = (bx2 - bx1) * (by2 - by1)
    area_p = (px2 - px1) * (py2 - py1)
    ov = inter / (area_b + area_p - inter)

    obj_iota = jax.lax.broadcasted_iota(jnp.int32, (n_obj, n_p), 0)
    col_iota = jax.lax.broadcasted_iota(jnp.int32, (n_obj, n_p), 1)

    ovmax_p = jnp.max(ov, axis=0, keepdims=True)
    ofep = jnp.min(jnp.where(ov == ovmax_p, obj_iota, n_obj),
                   axis=0, keepdims=True)
    rowmax = jnp.max(ov, axis=1, keepdims=True)
    pfeo = jnp.min(jnp.where(ov == rowmax, col_iota, n_p),
                   axis=1, keepdims=True)

    is_best = col_iota == pfeo
    j_assign = jnp.max(jnp.where(is_best, obj_iota, -1),
                       axis=0, keepdims=True)
    forced = j_assign >= 0
    ofep = jnp.where(forced, j_assign, ofep)
    ovfep = jnp.where(forced, 1.0, ovmax_p)

    sel = obj_iota == ofep
    lfp = jnp.sum(jnp.where(sel, lab, 0), axis=0, keepdims=True)
    tc = jnp.where(ovfep < _THRESHOLD, 0, lfp)
    pos = tc != 0
    posf = pos.astype(jnp.float32)
    npos_i = jnp.sum(posf, axis=1, keepdims=True)

    zf = jnp.zeros((), jnp.float32)
    g_x1 = jnp.sum(jnp.where(sel, bx1, zf), axis=0, keepdims=True)
    g_y1 = jnp.sum(jnp.where(sel, by1, zf), axis=0, keepdims=True)
    g_x2 = jnp.sum(jnp.where(sel, bx2, zf), axis=0, keepdims=True)
    g_y2 = jnp.sum(jnp.where(sel, by2, zf), axis=0, keepdims=True)

    gcx = (g_x1 + g_x2) * 0.5
    gcy = (g_y1 + g_y2) * 0.5
    gw = g_x2 - g_x1
    gh = g_y2 - g_y1
    t0 = (gcx - cx) / (w / 10.0)
    t1 = (gcy - cy) / (h / 10.0)
    t2 = jnp.log(gw / w) * 5.0
    t3 = jnp.log(gh / h) * 5.0

    plc = plocs_ref[...].reshape(4, n_p)
    ad = (jnp.abs(plc[0:1, :] - t0) + jnp.abs(plc[1:2, :] - t1)
          + jnp.abs(plc[2:3, :] - t2) + jnp.abs(plc[3:4, :] - t3))
    loc_i = jnp.sum(jnp.where(pos, ad, zf), axis=1, keepdims=True)

    st = scores_ref[...].reshape(n_cls, n_p)
    m = jnp.max(st, axis=0, keepdims=True)
    lse = jnp.log(jnp.sum(jnp.exp(st - m), axis=0, keepdims=True)) + m
    cls_iota = jax.lax.broadcasted_iota(jnp.int32, (n_cls, n_p), 0)
    sal = jnp.sum(jnp.where(cls_iota == tc, st, zf),
                  axis=0, keepdims=True)
    ce = lse - sal

    cp_i = jnp.sum(jnp.where(pos, ce, zf), axis=1, keepdims=True)
    v = jnp.where(pos, zf, ce)

    kf = _NEG_POS_RATIO * npos_i
    lo = jnp.zeros((1, 1), jnp.float32)
    hi = jnp.max(v, axis=1, keepdims=True)
    for _ in range(_BISECT_ITERS):
        mid = (lo + hi) * 0.5
        c = jnp.sum((v > mid).astype(jnp.float32), axis=1, keepdims=True)
        take_lo = c >= kf
        lo = jnp.where(take_lo, mid, lo)
        hi = jnp.where(take_lo, hi, mid)
    above = v > hi
    cnt = jnp.sum(above.astype(jnp.float32), axis=1, keepdims=True)
    hn_i = (jnp.sum(jnp.where(above, v, zf), axis=1, keepdims=True)
            + (kf - cnt) * hi)

    @pl.when(i == 0)
    def _():
        acc_loc[...] = loc_i
        acc_np[...] = npos_i
        acc_cp[...] = cp_i
        acc_hn[...] = hn_i

    @pl.when(i > 0)
    def _():
        acc_loc[...] = acc_loc[...] + loc_i
        acc_np[...] = acc_np[...] + npos_i
        acc_cp[...] = acc_cp[...] + cp_i
        acc_hn[...] = acc_hn[...] + hn_i

    @pl.when(i == n_b - 1)
    def _():
        npos_t = acc_np[...]
        conf = (acc_hn[...] + acc_cp[...]) / npos_t
        loc = acc_loc[...] / (npos_t * 4.0)
        out_ref[...] = conf + _ALPHA * loc


def kernel(predicted_locs, predicted_scores, boxes, labels, priors_cxcy):
    n_b, n_p, n_cls = predicted_scores.shape
    n_obj = boxes.shape[1]
    plocs_t = jnp.transpose(predicted_locs, (0, 2, 1))
    scores_t = jnp.transpose(predicted_scores, (0, 2, 1))
    priors_t = jnp.transpose(priors_cxcy, (1, 0))
    labels3 = labels.astype(jnp.int32)[..., None]

    out = pl.pallas_call(
        functools.partial(_mbl_kernel, n_b=n_b, n_obj=n_obj,
                          n_cls=n_cls, n_p=n_p),
        grid=(n_b,),
        in_specs=[
            pl.BlockSpec((1, 4, n_p), lambda i: (i, 0, 0)),
            pl.BlockSpec((1, n_cls, n_p), lambda i: (i, 0, 0)),
            pl.BlockSpec((1, n_obj, 4), lambda i: (i, 0, 0)),
            pl.BlockSpec((1, n_obj, 1), lambda i: (i, 0, 0)),
            pl.BlockSpec((4, n_p), lambda i: (0, 0)),
        ],
        out_specs=pl.BlockSpec((1, 1), lambda i: (0, 0)),
        out_shape=jax.ShapeDtypeStruct((1, 1), jnp.float32),
        scratch_shapes=[pltpu.VMEM((1, 1), jnp.float32)] * 4,
    )(plocs_t, scores_t, boxes, labels3, priors_t)
    return out[0, 0]


# speedup vs baseline: 1.6625x; 1.1920x over previous
import functools

import jax
import jax.numpy as jnp
from jax.experimental import pallas as pl
from jax.experimental.pallas import tpu as pltpu

_THRESHOLD = 0.5
_NEG_POS_RATIO = 3.0
_ALPHA = 1.0
_BISECT_ITERS = 20


def _mbl_kernel(plocs_ref, scores_ref, boxes_ref, boxlab_t_ref, priors_ref,
                out_ref, acc_loc, acc_np, acc_cp, acc_hn,
                *, n_b, n_obj, n_cls, n_p):
    i = pl.program_id(0)

    bv = boxes_ref[...].reshape(n_obj, 4)
    blt = boxlab_t_ref[...].reshape(5, n_obj)
    pt = priors_ref[...]
    cx = pt[0:1, :]
    cy = pt[1:2, :]
    w = pt[2:3, :]
    h = pt[3:4, :]
    px1 = cx - w * 0.5
    py1 = cy - h * 0.5
    px2 = cx + w * 0.5
    py2 = cy + h * 0.5

    bx1 = bv[:, 0:1]
    by1 = bv[:, 1:2]
    bx2 = bv[:, 2:3]
    by2 = bv[:, 3:4]

    iw = jnp.maximum(jnp.minimum(bx2, px2) - jnp.maximum(bx1, px1), 0.0)
    ih = jnp.maximum(jnp.minimum(by2, py2) - jnp.maximum(by1, py1), 0.0)
    inter = iw * ih
    area_b = (bx2 - bx1) * (by2 - by1)
    area_p = (px2 - px1) * (py2 - py1)
    ov = inter / (area_b + area_p - inter)

    obj_iota = jax.lax.broadcasted_iota(jnp.int32, (n_obj, n_p), 0)
    col_iota = jax.lax.broadcasted_iota(jnp.int32, (n_obj, n_p), 1)

    ovmax_p = jnp.max(ov, axis=0, keepdims=True)
    ofep = jnp.min(jnp.where(ov == ovmax_p, obj_iota, n_obj),
                   axis=0, keepdims=True)
    rowmax = jnp.max(ov, axis=1, keepdims=True)
    pfeo = jnp.min(jnp.where(ov == rowmax, col_iota, n_p),
                   axis=1, keepdims=True)

    is_best = col_iota == pfeo
    j_assign = jnp.max(jnp.where(is_best, obj_iota, -1),
                       axis=0, keepdims=True)
    forced = j_assign >= 0
    ofep = jnp.where(forced, j_assign, ofep)
    ovfep = jnp.where(forced, 1.0, ovmax_p)

    sel = obj_iota == ofep
    gat = jnp.dot(blt, sel.astype(jnp.float32),
                  preferred_element_type=jnp.float32)
    g_x1 = gat[0:1, :]
    g_y1 = gat[1:2, :]
    g_x2 = gat[2:3, :]
    g_y2 = gat[3:4, :]
    lfp = gat[4:5, :].astype(jnp.int32)
    tc = jnp.where(ovfep < _THRESHOLD, 0, lfp)
    pos = tc != 0
    posf = pos.astype(jnp.float32)
    npos_i = jnp.sum(posf, axis=1, keepdims=True)

    zf = jnp.zeros((), jnp.float32)

    gcx = (g_x1 + g_x2) * 0.5
    gcy = (g_y1 + g_y2) * 0.5
    gw = g_x2 - g_x1
    gh = g_y2 - g_y1
    t0 = (gcx - cx) / (w / 10.0)
    t1 = (gcy - cy) / (h / 10.0)
    t2 = jnp.log(gw / w) * 5.0
    t3 = jnp.log(gh / h) * 5.0

    plc = plocs_ref[...].reshape(4, n_p)
    ad = (jnp.abs(plc[0:1, :] - t0) + jnp.abs(plc[1:2, :] - t1)
          + jnp.abs(plc[2:3, :] - t2) + jnp.abs(plc[3:4, :] - t3))
    loc_i = jnp.sum(jnp.where(pos, ad, zf), axis=1, keepdims=True)

    st_n = scores_ref[...].reshape(n_p, n_cls)
    eye_r = jax.lax.broadcasted_iota(jnp.int32, (n_cls, n_cls), 0)
    eye_c = jax.lax.broadcasted_iota(jnp.int32, (n_cls, n_cls), 1)
    eye = (eye_r == eye_c).astype(jnp.float32)
    st = jax.lax.dot_general(eye, st_n, (((1,), (1,)), ((), ())),
                             preferred_element_type=jnp.float32)
    m = jnp.max(st, axis=0, keepdims=True)
    e = jnp.exp(st - m)
    ones_c = jnp.full((1, n_cls), 1.0, jnp.float32)
    esum = jnp.dot(ones_c, e, preferred_element_type=jnp.float32)
    lse = jnp.log(esum) + m
    cls_iota = jax.lax.broadcasted_iota(jnp.int32, (n_cls, n_p), 0)
    sal = jnp.sum(jnp.where(cls_iota == tc, st, zf),
                  axis=0, keepdims=True)
    ce = lse - sal

    cp_i = jnp.sum(jnp.where(pos, ce, zf), axis=1, keepdims=True)
    v = jnp.where(pos, zf, ce)

    kf = _NEG_POS_RATIO * npos_i
    lo = jnp.zeros((1, 1), jnp.float32)
    hi = jnp.max(v, axis=1, keepdims=True)
    for _ in range(_BISECT_ITERS):
        mid = (lo + hi) * 0.5
        c = jnp.sum((v > mid).astype(jnp.float32), axis=1, keepdims=True)
        take_lo = c >= kf
        lo = jnp.where(take_lo, mid, lo)
        hi = jnp.where(take_lo, hi, mid)
    above = v > hi
    cnt = jnp.sum(above.astype(jnp.float32), axis=1, keepdims=True)
    hn_i = (jnp.sum(jnp.where(above, v, zf), axis=1, keepdims=True)
            + (kf - cnt) * hi)

    @pl.when(i == 0)
    def _():
        acc_loc[...] = loc_i
        acc_np[...] = npos_i
        acc_cp[...] = cp_i
        acc_hn[...] = hn_i

    @pl.when(i > 0)
    def _():
        acc_loc[...] = acc_loc[...] + loc_i
        acc_np[...] = acc_np[...] + npos_i
        acc_cp[...] = acc_cp[...] + cp_i
        acc_hn[...] = acc_hn[...] + hn_i

    @pl.when(i == n_b - 1)
    def _():
        npos_t = acc_np[...]
        conf = (acc_hn[...] + acc_cp[...]) / npos_t
        loc = acc_loc[...] / (npos_t * 4.0)
        out_ref[...] = conf + _ALPHA * loc


def kernel(predicted_locs, predicted_scores, boxes, labels, priors_cxcy):
    n_b, n_p, n_cls = predicted_scores.shape
    n_obj = boxes.shape[1]
    plocs_t = jnp.transpose(predicted_locs, (0, 2, 1))
    priors_t = jnp.transpose(priors_cxcy, (1, 0))
    boxlab_t = jnp.concatenate(
        [jnp.transpose(boxes, (0, 2, 1)),
         labels.astype(jnp.float32)[:, None, :]], axis=1)

    out = pl.pallas_call(
        functools.partial(_mbl_kernel, n_b=n_b, n_obj=n_obj,
                          n_cls=n_cls, n_p=n_p),
        grid=(n_b,),
        in_specs=[
            pl.BlockSpec((1, 4, n_p), lambda i: (i, 0, 0)),
            pl.BlockSpec((1, n_p, n_cls), lambda i: (i, 0, 0)),
            pl.BlockSpec((1, n_obj, 4), lambda i: (i, 0, 0)),
            pl.BlockSpec((1, 5, n_obj), lambda i: (i, 0, 0)),
            pl.BlockSpec((4, n_p), lambda i: (0, 0)),
        ],
        out_specs=pl.BlockSpec((1, 1), lambda i: (0, 0)),
        out_shape=jax.ShapeDtypeStruct((1, 1), jnp.float32),
        scratch_shapes=[pltpu.VMEM((1, 1), jnp.float32)] * 4,
    )(plocs_t, predicted_scores, boxes, boxlab_t, priors_t)
    return out[0, 0]


# speedup vs baseline: 2.0086x; 1.2082x over previous
import functools

import jax
import jax.numpy as jnp
from jax.experimental import pallas as pl
from jax.experimental.pallas import tpu as pltpu

_THRESHOLD = 0.5
_NEG_POS_RATIO = 3.0
_ALPHA = 1.0
_BISECT_ITERS = 20
_IMGS_PER_STEP = 2


def _match_one(bv, blt, prior_rows, n_obj, n_cls, n_p, st_n):
    cx, cy, w, h, px1, py1, px2, py2 = prior_rows

    bx1 = bv[:, 0:1]
    by1 = bv[:, 1:2]
    bx2 = bv[:, 2:3]
    by2 = bv[:, 3:4]

    iw = jnp.maximum(jnp.minimum(bx2, px2) - jnp.maximum(bx1, px1), 0.0)
    ih = jnp.maximum(jnp.minimum(by2, py2) - jnp.maximum(by1, py1), 0.0)
    inter = iw * ih
    area_b = (bx2 - bx1) * (by2 - by1)
    area_p = (px2 - px1) * (py2 - py1)
    ov = inter / (area_b + area_p - inter)

    obj_iota = jax.lax.broadcasted_iota(jnp.int32, (n_obj, n_p), 0)
    col_iota = jax.lax.broadcasted_iota(jnp.int32, (n_obj, n_p), 1)

    ovmax_p = jnp.max(ov, axis=0, keepdims=True)
    ofep = jnp.min(jnp.where(ov == ovmax_p, obj_iota, n_obj),
                   axis=0, keepdims=True)
    rowmax = jnp.max(ov, axis=1, keepdims=True)
    pfeo = jnp.min(jnp.where(ov == rowmax, col_iota, n_p),
                   axis=1, keepdims=True)

    is_best = col_iota == pfeo
    j_assign = jnp.max(jnp.where(is_best, obj_iota, -1),
                       axis=0, keepdims=True)
    forced = j_assign >= 0
    ofep = jnp.where(forced, j_assign, ofep)
    ovfep = jnp.where(forced, 1.0, ovmax_p)

    sel = obj_iota == ofep
    gat = jnp.dot(blt, sel.astype(jnp.float32),
                  preferred_element_type=jnp.float32)
    g_x1 = gat[0:1, :]
    g_y1 = gat[1:2, :]
    g_x2 = gat[2:3, :]
    g_y2 = gat[3:4, :]
    lfp = gat[4:5, :].astype(jnp.int32)
    tc = jnp.where(ovfep < _THRESHOLD, 0, lfp)
    pos = tc != 0

    gcx = (g_x1 + g_x2) * 0.5
    gcy = (g_y1 + g_y2) * 0.5
    gw = g_x2 - g_x1
    gh = g_y2 - g_y1
    t0 = (gcx - cx) / (w / 10.0)
    t1 = (gcy - cy) / (h / 10.0)
    t2 = jnp.log(gw / w) * 5.0
    t3 = jnp.log(gh / h) * 5.0

    eye_r = jax.lax.broadcasted_iota(jnp.int32, (n_cls, n_cls), 0)
    eye_c = jax.lax.broadcasted_iota(jnp.int32, (n_cls, n_cls), 1)
    eye = (eye_r == eye_c).astype(jnp.float32)
    st = jax.lax.dot_general(eye, st_n, (((1,), (1,)), ((), ())),
                             preferred_element_type=jnp.float32)
    m = jnp.max(st, axis=0, keepdims=True)
    e = jnp.exp(st - m)
    ones_c = jnp.full((1, n_cls), 1.0, jnp.float32)
    esum = jnp.dot(ones_c, e, preferred_element_type=jnp.float32)
    lse = jnp.log(esum) + m
    zf = jnp.zeros((), jnp.float32)
    cls_iota = jax.lax.broadcasted_iota(jnp.int32, (n_cls, n_p), 0)
    sal = jnp.sum(jnp.where(cls_iota == tc, st, zf),
                  axis=0, keepdims=True)
    ce = lse - sal
    return pos, (t0, t1, t2, t3), ce


def _mbl_kernel(plocs_ref, scores_ref, boxes_ref, boxlab_t_ref, priors_ref,
                out_ref, acc_loc, acc_np, acc_cp, acc_hn,
                *, n_b, n_obj, n_cls, n_p, n_img):
    i = pl.program_id(0)
    zf = jnp.zeros((), jnp.float32)

    pt = priors_ref[...]
    cx = pt[0:1, :]
    cy = pt[1:2, :]
    w = pt[2:3, :]
    h = pt[3:4, :]
    prior_rows = (cx, cy, w, h,
                  cx - w * 0.5, cy - h * 0.5, cx + w * 0.5, cy + h * 0.5)

    bva = boxes_ref[...]
    blta = boxlab_t_ref[...]
    sta = scores_ref[...]
    plca = plocs_ref[...]

    pos_l, ad_l, ce_l = [], [], []
    for j in range(n_img):
        pos_j, (t0, t1, t2, t3), ce_j = _match_one(
            bva[j], blta[j], prior_rows, n_obj, n_cls, n_p, sta[j])
        plc = plca[j]
        ad_j = (jnp.abs(plc[0:1, :] - t0) + jnp.abs(plc[1:2, :] - t1)
                + jnp.abs(plc[2:3, :] - t2) + jnp.abs(plc[3:4, :] - t3))
        pos_l.append(pos_j.astype(jnp.float32))
        ad_l.append(ad_j)
        ce_l.append(ce_j)

    posf = jnp.concatenate(pos_l, axis=0)
    ad = jnp.concatenate(ad_l, axis=0)
    ce = jnp.concatenate(ce_l, axis=0)
    npos = jnp.sum(posf, axis=1, keepdims=True)
    loc = jnp.sum(posf * ad, axis=1, keepdims=True)
    cp = jnp.sum(posf * ce, axis=1, keepdims=True)
    v = ce * (1.0 - posf)

    kf = _NEG_POS_RATIO * npos
    lo = jnp.zeros((n_img, 1), jnp.float32)
    hi = jnp.max(v, axis=1, keepdims=True)
    for _ in range(_BISECT_ITERS):
        mid = (lo + hi) * 0.5
        c = jnp.sum((v > mid).astype(jnp.float32), axis=1, keepdims=True)
        take_lo = c >= kf
        lo = jnp.where(take_lo, mid, lo)
        hi = jnp.where(take_lo, hi, mid)
    above = v > hi
    cnt = jnp.sum(above.astype(jnp.float32), axis=1, keepdims=True)
    hn = (jnp.sum(jnp.where(above, v, zf), axis=1, keepdims=True)
          + (kf - cnt) * hi)

    loc_s = jnp.sum(loc, axis=0, keepdims=True)
    np_s = jnp.sum(npos, axis=0, keepdims=True)
    cp_s = jnp.sum(cp, axis=0, keepdims=True)
    hn_s = jnp.sum(hn, axis=0, keepdims=True)

    @pl.when(i == 0)
    def _():
        acc_loc[...] = loc_s
        acc_np[...] = np_s
        acc_cp[...] = cp_s
        acc_hn[...] = hn_s

    @pl.when(i > 0)
    def _():
        acc_loc[...] = acc_loc[...] + loc_s
        acc_np[...] = acc_np[...] + np_s
        acc_cp[...] = acc_cp[...] + cp_s
        acc_hn[...] = acc_hn[...] + hn_s

    @pl.when(i == (n_b // n_img) - 1)
    def _():
        npos_t = acc_np[...]
        conf = (acc_hn[...] + acc_cp[...]) / npos_t
        loc_t = acc_loc[...] / (npos_t * 4.0)
        out_ref[...] = conf + _ALPHA * loc_t


def kernel(predicted_locs, predicted_scores, boxes, labels, priors_cxcy):
    n_b, n_p, n_cls = predicted_scores.shape
    n_obj = boxes.shape[1]
    n_img = _IMGS_PER_STEP
    plocs_t = jnp.transpose(predicted_locs, (0, 2, 1))
    priors_t = jnp.transpose(priors_cxcy, (1, 0))
    boxlab_t = jnp.concatenate(
        [jnp.transpose(boxes, (0, 2, 1)),
         labels.astype(jnp.float32)[:, None, :]], axis=1)

    out = pl.pallas_call(
        functools.partial(_mbl_kernel, n_b=n_b, n_obj=n_obj,
                          n_cls=n_cls, n_p=n_p, n_img=n_img),
        grid=(n_b // n_img,),
        in_specs=[
            pl.BlockSpec((n_img, 4, n_p), lambda i: (i, 0, 0)),
            pl.BlockSpec((n_img, n_p, n_cls), lambda i: (i, 0, 0)),
            pl.BlockSpec((n_img, n_obj, 4), lambda i: (i, 0, 0)),
            pl.BlockSpec((n_img, 5, n_obj), lambda i: (i, 0, 0)),
            pl.BlockSpec((4, n_p), lambda i: (0, 0)),
        ],
        out_specs=pl.BlockSpec((1, 1), lambda i: (0, 0)),
        out_shape=jax.ShapeDtypeStruct((1, 1), jnp.float32),
        scratch_shapes=[pltpu.VMEM((1, 1), jnp.float32)] * 4,
    )(plocs_t, predicted_scores, boxes, boxlab_t, priors_t)
    return out[0, 0]


# speedup vs baseline: 2.1957x; 1.0931x over previous
import functools

import jax
import jax.numpy as jnp
from jax.experimental import pallas as pl
from jax.experimental.pallas import tpu as pltpu

_THRESHOLD = 0.5
_NEG_POS_RATIO = 3.0
_ALPHA = 1.0
_BISECT_ITERS = 20
_IMGS_PER_STEP = 4


def _match_one(bv, blt, prior_rows, n_obj, n_cls, n_p, st_n):
    cx, cy, w, h, px1, py1, px2, py2 = prior_rows

    bx1 = bv[:, 0:1]
    by1 = bv[:, 1:2]
    bx2 = bv[:, 2:3]
    by2 = bv[:, 3:4]

    iw = jnp.maximum(jnp.minimum(bx2, px2) - jnp.maximum(bx1, px1), 0.0)
    ih = jnp.maximum(jnp.minimum(by2, py2) - jnp.maximum(by1, py1), 0.0)
    inter = iw * ih
    area_b = (bx2 - bx1) * (by2 - by1)
    area_p = (px2 - px1) * (py2 - py1)
    ov = inter / (area_b + area_p - inter)

    obj_iota = jax.lax.broadcasted_iota(jnp.int32, (n_obj, n_p), 0)
    col_iota = jax.lax.broadcasted_iota(jnp.int32, (n_obj, n_p), 1)

    ovmax_p = jnp.max(ov, axis=0, keepdims=True)
    ofep = jnp.min(jnp.where(ov == ovmax_p, obj_iota, n_obj),
                   axis=0, keepdims=True)
    rowmax = jnp.max(ov, axis=1, keepdims=True)
    pfeo = jnp.min(jnp.where(ov == rowmax, col_iota, n_p),
                   axis=1, keepdims=True)

    is_best = col_iota == pfeo
    j_assign = jnp.max(jnp.where(is_best, obj_iota, -1),
                       axis=0, keepdims=True)
    forced = j_assign >= 0
    ofep = jnp.where(forced, j_assign, ofep)
    ovfep = jnp.where(forced, 1.0, ovmax_p)

    sel = obj_iota == ofep
    gat = jnp.dot(blt, sel.astype(jnp.float32),
                  preferred_element_type=jnp.float32)
    g_x1 = gat[0:1, :]
    g_y1 = gat[1:2, :]
    g_x2 = gat[2:3, :]
    g_y2 = gat[3:4, :]
    lfp = gat[4:5, :].astype(jnp.int32)
    tc = jnp.where(ovfep < _THRESHOLD, 0, lfp)
    pos = tc != 0

    gcx = (g_x1 + g_x2) * 0.5
    gcy = (g_y1 + g_y2) * 0.5
    gw = g_x2 - g_x1
    gh = g_y2 - g_y1
    t0 = (gcx - cx) / (w / 10.0)
    t1 = (gcy - cy) / (h / 10.0)
    t2 = jnp.log(gw / w) * 5.0
    t3 = jnp.log(gh / h) * 5.0

    eye_r = jax.lax.broadcasted_iota(jnp.int32, (n_cls, n_cls), 0)
    eye_c = jax.lax.broadcasted_iota(jnp.int32, (n_cls, n_cls), 1)
    eye = (eye_r == eye_c).astype(jnp.float32)
    st = jax.lax.dot_general(eye, st_n, (((1,), (1,)), ((), ())),
                             preferred_element_type=jnp.float32)
    m = jnp.max(st, axis=0, keepdims=True)
    e = jnp.exp(st - m)
    ones_c = jnp.full((1, n_cls), 1.0, jnp.float32)
    esum = jnp.dot(ones_c, e, preferred_element_type=jnp.float32)
    lse = jnp.log(esum) + m
    zf = jnp.zeros((), jnp.float32)
    cls_iota = jax.lax.broadcasted_iota(jnp.int32, (n_cls, n_p), 0)
    sal = jnp.sum(jnp.where(cls_iota == tc, st, zf),
                  axis=0, keepdims=True)
    ce = lse - sal
    return pos, (t0, t1, t2, t3), ce


def _mbl_kernel(plocs_ref, scores_ref, boxes_ref, boxlab_t_ref, priors_ref,
                out_ref, acc_loc, acc_np, acc_cp, acc_hn,
                *, n_b, n_obj, n_cls, n_p, n_img):
    i = pl.program_id(0)
    zf = jnp.zeros((), jnp.float32)

    pt = priors_ref[...]
    cx = pt[0:1, :]
    cy = pt[1:2, :]
    w = pt[2:3, :]
    h = pt[3:4, :]
    prior_rows = (cx, cy, w, h,
                  cx - w * 0.5, cy - h * 0.5, cx + w * 0.5, cy + h * 0.5)

    bva = boxes_ref[...]
    blta = boxlab_t_ref[...]
    sta = scores_ref[...]
    plca = plocs_ref[...]

    pos_l, ad_l, ce_l = [], [], []
    for j in range(n_img):
        pos_j, (t0, t1, t2, t3), ce_j = _match_one(
            bva[j], blta[j], prior_rows, n_obj, n_cls, n_p, sta[j])
        plc = plca[j]
        ad_j = (jnp.abs(plc[0:1, :] - t0) + jnp.abs(plc[1:2, :] - t1)
                + jnp.abs(plc[2:3, :] - t2) + jnp.abs(plc[3:4, :] - t3))
        pos_l.append(pos_j.astype(jnp.float32))
        ad_l.append(ad_j)
        ce_l.append(ce_j)

    posf = jnp.concatenate(pos_l, axis=0)
    ad = jnp.concatenate(ad_l, axis=0)
    ce = jnp.concatenate(ce_l, axis=0)
    npos = jnp.sum(posf, axis=1, keepdims=True)
    loc = jnp.sum(posf * ad, axis=1, keepdims=True)
    cp = jnp.sum(posf * ce, axis=1, keepdims=True)
    v = ce * (1.0 - posf)

    kf = _NEG_POS_RATIO * npos
    lo = jnp.zeros((n_img, 1), jnp.float32)
    hi = jnp.max(v, axis=1, keepdims=True)
    for _ in range(_BISECT_ITERS):
        mid = (lo + hi) * 0.5
        c = jnp.sum((v > mid).astype(jnp.float32), axis=1, keepdims=True)
        take_lo = c >= kf
        lo = jnp.where(take_lo, mid, lo)
        hi = jnp.where(take_lo, hi, mid)
    above = v > hi
    cnt = jnp.sum(above.astype(jnp.float32), axis=1, keepdims=True)
    hn = (jnp.sum(jnp.where(above, v, zf), axis=1, keepdims=True)
          + (kf - cnt) * hi)

    loc_s = jnp.sum(loc, axis=0, keepdims=True)
    np_s = jnp.sum(npos, axis=0, keepdims=True)
    cp_s = jnp.sum(cp, axis=0, keepdims=True)
    hn_s = jnp.sum(hn, axis=0, keepdims=True)

    @pl.when(i == 0)
    def _():
        acc_loc[...] = loc_s
        acc_np[...] = np_s
        acc_cp[...] = cp_s
        acc_hn[...] = hn_s

    @pl.when(i > 0)
    def _():
        acc_loc[...] = acc_loc[...] + loc_s
        acc_np[...] = acc_np[...] + np_s
        acc_cp[...] = acc_cp[...] + cp_s
        acc_hn[...] = acc_hn[...] + hn_s

    @pl.when(i == (n_b // n_img) - 1)
    def _():
        npos_t = acc_np[...]
        conf = (acc_hn[...] + acc_cp[...]) / npos_t
        loc_t = acc_loc[...] / (npos_t * 4.0)
        out_ref[...] = conf + _ALPHA * loc_t


def kernel(predicted_locs, predicted_scores, boxes, labels, priors_cxcy):
    n_b, n_p, n_cls = predicted_scores.shape
    n_obj = boxes.shape[1]
    n_img = _IMGS_PER_STEP
    plocs_t = jnp.transpose(predicted_locs, (0, 2, 1))
    priors_t = jnp.transpose(priors_cxcy, (1, 0))
    boxlab_t = jnp.concatenate(
        [jnp.transpose(boxes, (0, 2, 1)),
         labels.astype(jnp.float32)[:, None, :]], axis=1)

    out = pl.pallas_call(
        functools.partial(_mbl_kernel, n_b=n_b, n_obj=n_obj,
                          n_cls=n_cls, n_p=n_p, n_img=n_img),
        grid=(n_b // n_img,),
        in_specs=[
            pl.BlockSpec((n_img, 4, n_p), lambda i: (i, 0, 0)),
            pl.BlockSpec((n_img, n_p, n_cls), lambda i: (i, 0, 0)),
            pl.BlockSpec((n_img, n_obj, 4), lambda i: (i, 0, 0)),
            pl.BlockSpec((n_img, 5, n_obj), lambda i: (i, 0, 0)),
            pl.BlockSpec((4, n_p), lambda i: (0, 0)),
        ],
        out_specs=pl.BlockSpec((1, 1), lambda i: (0, 0)),
        out_shape=jax.ShapeDtypeStruct((1, 1), jnp.float32),
        scratch_shapes=[pltpu.VMEM((1, 1), jnp.float32)] * 4,
    )(plocs_t, predicted_scores, boxes, boxlab_t, priors_t)
    return out[0, 0]


# speedup vs baseline: 2.2182x; 1.0103x over previous
import functools

import jax
import jax.numpy as jnp
from jax.experimental import pallas as pl
from jax.experimental.pallas import tpu as pltpu

_THRESHOLD = 0.5
_NEG_POS_RATIO = 3.0
_ALPHA = 1.0
_BISECT_ITERS = 20
_IMGS_PER_STEP = 4


def _match_one(bv, labf, prior_rows, n_obj, n_cls, n_p, st_n):
    cx, cy, w, h, px1, py1, px2, py2 = prior_rows

    bx1 = bv[:, 0:1]
    by1 = bv[:, 1:2]
    bx2 = bv[:, 2:3]
    by2 = bv[:, 3:4]

    iw = jnp.maximum(jnp.minimum(bx2, px2) - jnp.maximum(bx1, px1), 0.0)
    ih = jnp.maximum(jnp.minimum(by2, py2) - jnp.maximum(by1, py1), 0.0)
    inter = iw * ih
    area_b = (bx2 - bx1) * (by2 - by1)
    area_p = (px2 - px1) * (py2 - py1)
    ov = inter / (area_b + area_p - inter)

    obj_iota = jax.lax.broadcasted_iota(jnp.int32, (n_obj, n_p), 0)
    col_iota = jax.lax.broadcasted_iota(jnp.int32, (n_obj, n_p), 1)

    ovmax_p = jnp.max(ov, axis=0, keepdims=True)
    ofep = jnp.min(jnp.where(ov == ovmax_p, obj_iota, n_obj),
                   axis=0, keepdims=True)
    rowmax = jnp.max(ov, axis=1, keepdims=True)
    pfeo = jnp.min(jnp.where(ov == rowmax, col_iota, n_p),
                   axis=1, keepdims=True)

    is_best = col_iota == pfeo
    j_assign = jnp.max(jnp.where(is_best, obj_iota, -1),
                       axis=0, keepdims=True)
    forced = j_assign >= 0
    ofep = jnp.where(forced, j_assign, ofep)
    ovfep = jnp.where(forced, 1.0, ovmax_p)

    sel = obj_iota == ofep
    bv5 = jnp.concatenate([bv, labf], axis=1)
    gat = jax.lax.dot_general(bv5, sel.astype(jnp.float32),
                              (((0,), (0,)), ((), ())),
                              preferred_element_type=jnp.float32)
    g_x1 = gat[0:1, :]
    g_y1 = gat[1:2, :]
    g_x2 = gat[2:3, :]
    g_y2 = gat[3:4, :]
    lfp = gat[4:5, :].astype(jnp.int32)
    tc = jnp.where(ovfep < _THRESHOLD, 0, lfp)
    pos = tc != 0

    gcx = (g_x1 + g_x2) * 0.5
    gcy = (g_y1 + g_y2) * 0.5
    gw = g_x2 - g_x1
    gh = g_y2 - g_y1
    t0 = (gcx - cx) / (w / 10.0)
    t1 = (gcy - cy) / (h / 10.0)
    t2 = jnp.log(gw / w) * 5.0
    t3 = jnp.log(gh / h) * 5.0

    eye_r = jax.lax.broadcasted_iota(jnp.int32, (n_cls, n_cls), 0)
    eye_c = jax.lax.broadcasted_iota(jnp.int32, (n_cls, n_cls), 1)
    eye = (eye_r == eye_c).astype(jnp.float32)
    st = jax.lax.dot_general(eye, st_n, (((1,), (1,)), ((), ())),
                             preferred_element_type=jnp.float32)
    m = jnp.max(st, axis=0, keepdims=True)
    e = jnp.exp(st - m)
    ones_c = jnp.full((1, n_cls), 1.0, jnp.float32)
    esum = jnp.dot(ones_c, e, preferred_element_type=jnp.float32)
    lse = jnp.log(esum) + m
    zf = jnp.zeros((), jnp.float32)
    cls_iota = jax.lax.broadcasted_iota(jnp.int32, (n_cls, n_p), 0)
    sal = jnp.sum(jnp.where(cls_iota == tc, st, zf),
                  axis=0, keepdims=True)
    ce = lse - sal
    return pos, (t0, t1, t2, t3), ce


def _mbl_kernel(plocs_ref, scores_ref, boxes_ref, labels_ref, priors_ref,
                out_ref, acc_loc, acc_np, acc_cp, acc_hn,
                *, n_b, n_obj, n_cls, n_p, n_img):
    i = pl.program_id(0)
    zf = jnp.zeros((), jnp.float32)

    pt = priors_ref[...]
    cx = pt[0:1, :]
    cy = pt[1:2, :]
    w = pt[2:3, :]
    h = pt[3:4, :]
    prior_rows = (cx, cy, w, h,
                  cx - w * 0.5, cy - h * 0.5, cx + w * 0.5, cy + h * 0.5)

    bva = boxes_ref[...]
    labfa = labels_ref[...]
    sta = scores_ref[...]
    plca = plocs_ref[...]

    pos_l, ad_l, ce_l = [], [], []
    for j in range(n_img):
        pos_j, (t0, t1, t2, t3), ce_j = _match_one(
            bva[j], labfa[j], prior_rows, n_obj, n_cls, n_p, sta[j])
        plc = plca[j]
        ad_j = (jnp.abs(plc[0:1, :] - t0) + jnp.abs(plc[1:2, :] - t1)
                + jnp.abs(plc[2:3, :] - t2) + jnp.abs(plc[3:4, :] - t3))
        pos_l.append(pos_j.astype(jnp.float32))
        ad_l.append(ad_j)
        ce_l.append(ce_j)

    posf = jnp.concatenate(pos_l, axis=0)
    ad = jnp.concatenate(ad_l, axis=0)
    ce = jnp.concatenate(ce_l, axis=0)
    npos = jnp.sum(posf, axis=1, keepdims=True)
    loc = jnp.sum(posf * ad, axis=1, keepdims=True)
    cp = jnp.sum(posf * ce, axis=1, keepdims=True)
    v = ce * (1.0 - posf)

    kf = _NEG_POS_RATIO * npos
    lo = jnp.zeros((n_img, 1), jnp.float32)
    hi = jnp.max(v, axis=1, keepdims=True)
    for _ in range(_BISECT_ITERS):
        mid = (lo + hi) * 0.5
        c = jnp.sum((v > mid).astype(jnp.float32), axis=1, keepdims=True)
        take_lo = c >= kf
        lo = jnp.where(take_lo, mid, lo)
        hi = jnp.where(take_lo, hi, mid)
    above = v > hi
    cnt = jnp.sum(above.astype(jnp.float32), axis=1, keepdims=True)
    hn = (jnp.sum(jnp.where(above, v, zf), axis=1, keepdims=True)
          + (kf - cnt) * hi)

    loc_s = jnp.sum(loc, axis=0, keepdims=True)
    np_s = jnp.sum(npos, axis=0, keepdims=True)
    cp_s = jnp.sum(cp, axis=0, keepdims=True)
    hn_s = jnp.sum(hn, axis=0, keepdims=True)

    @pl.when(i == 0)
    def _():
        acc_loc[...] = loc_s
        acc_np[...] = np_s
        acc_cp[...] = cp_s
        acc_hn[...] = hn_s

    @pl.when(i > 0)
    def _():
        acc_loc[...] = acc_loc[...] + loc_s
        acc_np[...] = acc_np[...] + np_s
        acc_cp[...] = acc_cp[...] + cp_s
        acc_hn[...] = acc_hn[...] + hn_s

    @pl.when(i == (n_b // n_img) - 1)
    def _():
        npos_t = acc_np[...]
        conf = (acc_hn[...] + acc_cp[...]) / npos_t
        loc_t = acc_loc[...] / (npos_t * 4.0)
        out_ref[...] = conf + _ALPHA * loc_t


def kernel(predicted_locs, predicted_scores, boxes, labels, priors_cxcy):
    n_b, n_p, n_cls = predicted_scores.shape
    n_obj = boxes.shape[1]
    n_img = _IMGS_PER_STEP
    plocs_t = jnp.transpose(predicted_locs, (0, 2, 1))
    priors_t = jnp.transpose(priors_cxcy, (1, 0))
    labels_f = labels.astype(jnp.float32)[..., None]

    out = pl.pallas_call(
        functools.partial(_mbl_kernel, n_b=n_b, n_obj=n_obj,
                          n_cls=n_cls, n_p=n_p, n_img=n_img),
        grid=(n_b // n_img,),
        in_specs=[
            pl.BlockSpec((n_img, 4, n_p), lambda i: (i, 0, 0)),
            pl.BlockSpec((n_img, n_p, n_cls), lambda i: (i, 0, 0)),
            pl.BlockSpec((n_img, n_obj, 4), lambda i: (i, 0, 0)),
            pl.BlockSpec((n_img, n_obj, 1), lambda i: (i, 0, 0)),
            pl.BlockSpec((4, n_p), lambda i: (0, 0)),
        ],
        out_specs=pl.BlockSpec((1, 1), lambda i: (0, 0)),
        out_shape=jax.ShapeDtypeStruct((1, 1), jnp.float32),
        scratch_shapes=[pltpu.VMEM((1, 1), jnp.float32)] * 4,
    )(plocs_t, predicted_scores, boxes, labels_f, priors_t)
    return out[0, 0]
